# baseline (speedup 1.0000x reference)
import jax
import jax.numpy as jnp
from jax.experimental import pallas as pl
from jax.experimental.pallas import tpu as pltpu

_NUM_CLASSES = 8192
_Z_DIM = 256


def _copy_body(a_hbm, o_hbm, buf, in_sem, out_sem):
    pltpu.make_async_copy(a_hbm, buf, in_sem).start()
    pltpu.make_async_copy(a_hbm, buf, in_sem).wait()
    pltpu.make_async_copy(buf, o_hbm, out_sem).start()
    pltpu.make_async_copy(buf, o_hbm, out_sem).wait()


def kernel(_, anchor):
    return pl.pallas_call(
        _copy_body,
        in_specs=[pl.BlockSpec(memory_space=pl.ANY)],
        out_specs=pl.BlockSpec(memory_space=pl.ANY),
        out_shape=jax.ShapeDtypeStruct((_NUM_CLASSES, _Z_DIM), jnp.float32),
        scratch_shapes=[
            pltpu.VMEM((_NUM_CLASSES, _Z_DIM), jnp.float32),
            pltpu.SemaphoreType.DMA,
            pltpu.SemaphoreType.DMA,
        ],
    )(anchor)
